# Initial kernel scaffold; baseline (speedup 1.0000x reference)
import jax
import jax.numpy as jnp
from jax.experimental import pallas as pl


def kernel(in_capsules, conv_w, conv_b, ln_gamma, ln_beta):
    raise NotImplementedError("write your pallas kernel here")


# speedup vs baseline: 114.4774x; 114.4774x over previous
import functools

import jax
import jax.numpy as jnp
from jax.experimental import pallas as pl

_IN_CAPS = 16
_IN_DIM = 16
_OUT_CAPS = 8
_OUT_DIM = 16
_COUT = _OUT_CAPS * _OUT_DIM
_CIN = _IN_DIM * 4
_HO = 16
_WO = 16
_NPIX = _HO * _WO
_K = 9 * _CIN + 8


def _conv_squash_body(x_ref, w_ref, o_ref):
    w = w_ref[...]
    nb = x_ref.shape[0]
    for b in range(nb):
        xs = jnp.sum(x_ref[b], axis=0)
        planes = []
        for dh in range(3):
            for dw in range(3):
                rp = 0 if dh == 1 else 1
                cp = 0 if dw == 1 else 1
                a = xs[2 * rp + cp]
                if dh == 0:
                    a = jnp.concatenate(
                        [jnp.zeros((_CIN, _WO), jnp.float32), a[:, :-_WO]], axis=1)
                if dw == 0:
                    a = jnp.concatenate(
                        [jnp.zeros((_CIN, 1), jnp.float32), a[:, :-1]], axis=1)
                    lane = jax.lax.broadcasted_iota(jnp.int32, (_CIN, _NPIX), 1)
                    a = jnp.where(lane % _WO == 0, 0.0, a)
                planes.append(a)
        planes.append(jnp.ones((8, _NPIX), jnp.float32))
        col = jnp.concatenate(planes, axis=0)
        y = jax.lax.dot_general(
            w, col, (((1,), (0,)), ((), ())),
            preferred_element_type=jnp.float32)
        ys = y.reshape(4, _COUT, _NPIX)
        n2 = jnp.sum(ys * ys, axis=0, keepdims=True)
        norm = jnp.sqrt(n2)
        scale = n2 / (1.0 + n2) / (norm + 1e-8)
        o_ref[b] = ys * scale


@functools.partial(jax.jit, static_argnames=())
def kernel(in_capsules, conv_w, conv_b, ln_gamma, ln_beta):
    del ln_gamma, ln_beta
    nb = in_capsules.shape[0]

    rotated = []
    for s in range(4):
        wr = jnp.rot90(conv_w, k=s, axes=(-2, -1))
        wr = jnp.roll(wr, shift=s, axis=2)
        rotated.append(wr)
    wfull = jnp.stack(rotated, axis=0)
    w_sm = wfull.reshape(4 * _COUT, _CIN, 3, 3)
    wmat = jnp.transpose(w_sm, (0, 2, 3, 1)).reshape(4 * _COUT, 9 * _CIN)
    bias_sm = jnp.tile(conv_b * float(_IN_CAPS), (4,))
    wext = jnp.concatenate(
        [wmat, jnp.tile((bias_sm / 8.0)[:, None], (1, 8))], axis=1)

    x = in_capsules.reshape(nb, _IN_CAPS, _CIN, _HO, 2, _WO, 2)
    xpar = jnp.transpose(x, (0, 1, 4, 6, 2, 3, 5)).reshape(
        nb, _IN_CAPS, 4, _CIN, _NPIX)

    out = pl.pallas_call(
        _conv_squash_body,
        out_shape=jax.ShapeDtypeStruct((nb, 4, _COUT, _NPIX), jnp.float32),
    )(xpar, wext)

    return out.reshape(nb, 4, _OUT_CAPS, _OUT_DIM, _HO, _WO).transpose(
        0, 2, 3, 1, 4, 5)
